# baseline (speedup 1.0000x reference)
import functools

import jax
import jax.numpy as jnp
from jax import lax
from jax.experimental import pallas as pl
from jax.experimental.pallas import tpu as pltpu
from jax.experimental.pallas import tpu_sc as plsc

B = 16384
L = 64
V = 1000000
E = 64
H = 128
IMG = 2048
C = 1000


_MEAN_ROWS = 8000


def _row_mean_body(tab_ref, out_ref):
    out_ref[...] = jnp.mean(tab_ref[...], axis=1, keepdims=True)


def _row_means(table):
    grid = V // _MEAN_ROWS
    out = pl.pallas_call(
        _row_mean_body,
        grid=(grid,),
        in_specs=[pl.BlockSpec((_MEAN_ROWS, E), lambda i: (i, 0))],
        out_specs=pl.BlockSpec((_MEAN_ROWS, 1), lambda i: (i, 0)),
        out_shape=jax.ShapeDtypeStruct((V, 1), jnp.float32),
    )(table)
    return out.reshape(V)


_NC = 2
_NS = 16
_NW = _NC * _NS
_N_IDX = B * L
_PER_W = _N_IDX // _NW
_CHUNK = 128
_NCH = _PER_W // _CHUNK
_GRP = 8


def _gather_body(means_hbm, idx_hbm, out_hbm, idx_v, rows_v, sem):
    wid = lax.axis_index("s") * _NC + lax.axis_index("c")
    pltpu.sync_copy(idx_hbm.at[wid], idx_v)

    for b in range(_GRP):
        pltpu.async_copy(means_hbm.at[idx_v.at[b]], rows_v.at[b], sem)

    def body(g, carry):
        nxt = (g + 1) * _GRP
        cur = g * _GRP
        for b in range(_GRP):
            pltpu.async_copy(means_hbm.at[idx_v.at[nxt + b]], rows_v.at[nxt + b], sem)
        for b in range(_GRP):
            pltpu.make_async_copy(
                means_hbm.at[pl.ds(0, _CHUNK)], rows_v.at[cur + b], sem
            ).wait()
        return carry

    lax.fori_loop(0, _NCH // _GRP - 1, body, 0)

    last = (_NCH // _GRP - 1) * _GRP
    for b in range(_GRP):
        pltpu.make_async_copy(
            means_hbm.at[pl.ds(0, _CHUNK)], rows_v.at[last + b], sem
        ).wait()

    pltpu.sync_copy(rows_v, out_hbm.at[wid])


def _gather_means(means, idx_flat):
    idx3 = idx_flat.reshape(_NW, _NCH, _CHUNK)
    k = pl.kernel(
        _gather_body,
        out_type=jax.ShapeDtypeStruct((_NW, _NCH, _CHUNK), jnp.float32),
        mesh=plsc.VectorSubcoreMesh(core_axis_name="c", subcore_axis_name="s"),
        scratch_types=[
            pltpu.VMEM((_NCH, _CHUNK), jnp.int32),
            pltpu.VMEM((_NCH, _CHUNK), jnp.float32),
            pltpu.SemaphoreType.DMA,
        ],
    )
    return k(means, idx3).reshape(B, L)


_RB = 1024


def _dense_body(qf_ref, img_ref, q2h_wt_ref, q2h_b_ref, i2h_wt_ref, i2h_b_ref,
                sc_wt_ref, sc_b_ref, out_ref):
    h_q = jnp.dot(qf_ref[...], q2h_wt_ref[...],
                  preferred_element_type=jnp.float32) + q2h_b_ref[...]
    h_i = jnp.dot(img_ref[...], i2h_wt_ref[...],
                  preferred_element_type=jnp.float32) + i2h_b_ref[...]
    comb = h_q * h_i
    out_ref[...] = jnp.dot(comb, sc_wt_ref[...],
                           preferred_element_type=jnp.float32) + sc_b_ref[...]


def _dense(qf, image_emb, q2h_W, q2h_b, i2h_W, i2h_b, sc_W, sc_b):
    grid = B // _RB
    return pl.pallas_call(
        _dense_body,
        grid=(grid,),
        in_specs=[
            pl.BlockSpec((_RB, L), lambda i: (i, 0)),
            pl.BlockSpec((_RB, IMG), lambda i: (i, 0)),
            pl.BlockSpec((L, H), lambda i: (0, 0)),
            pl.BlockSpec((1, H), lambda i: (0, 0)),
            pl.BlockSpec((IMG, H), lambda i: (0, 0)),
            pl.BlockSpec((1, H), lambda i: (0, 0)),
            pl.BlockSpec((H, C), lambda i: (0, 0)),
            pl.BlockSpec((1, C), lambda i: (0, 0)),
        ],
        out_specs=pl.BlockSpec((_RB, C), lambda i: (i, 0)),
        out_shape=jax.ShapeDtypeStruct((B, C), jnp.float32),
    )(qf, image_emb, q2h_W.T, q2h_b.reshape(1, H), i2h_W.T,
      i2h_b.reshape(1, H), sc_W.T, sc_b.reshape(1, C))


def kernel(questions_idxs, image_emb, embs_weight, q2h_W, q2h_b, i2h_W, i2h_b,
           sc_W, sc_b):
    means = _row_means(embs_weight)
    idx_flat = questions_idxs.astype(jnp.int32).reshape(-1)
    qf = _gather_means(means, idx_flat)
    return _dense(qf, image_emb, q2h_W, q2h_b, i2h_W, i2h_b, sc_W, sc_b)
